# baseline (speedup 1.0000x reference)
import functools

import jax
import jax.numpy as jnp
from jax import lax
from jax.experimental import pallas as pl
from jax.experimental.pallas import tpu as pltpu
from jax.experimental.pallas import tpu_sc as plsc

_INFO = plsc.get_sparse_core_info()
_NC = _INFO.num_cores
_NS = _INFO.num_subcores
_NW = _NC * _NS
_L = 128


def _gather_impl(t_steps, b, d, table, idx_tb):
    bw = b // _NW
    assert b % _NW == 0 and bw % _L == 0 and t_steps % 2 == 0
    nj = bw // _L

    mesh = plsc.VectorSubcoreMesh(core_axis_name="c", subcore_axis_name="s")

    @functools.partial(
        pl.kernel,
        mesh=mesh,
        compiler_params=pltpu.CompilerParams(
            use_tc_tiling_on_sc=False, needs_layout_passes=False
        ),
        out_type=jax.ShapeDtypeStruct((t_steps, d, b), jnp.float32),
        scratch_types=[
            pltpu.VMEM((2, bw), jnp.int32),
            pltpu.VMEM((2, bw, d), jnp.float32),
            pltpu.VMEM((d, bw), jnp.float32),
            pltpu.SemaphoreType.DMA,
            pltpu.SemaphoreType.DMA,
        ],
    )
    def k(table_hbm, idx_hbm, out_hbm, idx_v, rows_v, trans_v, sem0, sem1):
        sems = (sem0, sem1)
        wid = lax.axis_index("s") * _NC + lax.axis_index("c")
        b0 = wid * bw
        lanes = lax.iota(jnp.int32, 16)

        def start_chunk(t, s):
            pltpu.sync_copy(idx_hbm.at[t, pl.ds(b0, bw)], idx_v.at[s])
            for j in range(nj):
                pltpu.make_async_copy(
                    table_hbm.at[idx_v.at[s].at[pl.ds(j * _L, _L)]],
                    rows_v.at[s].at[pl.ds(j * _L, _L)],
                    sems[s],
                ).start()

        def finish_chunk(t, s):
            for j in range(nj):
                pltpu.make_async_copy(
                    table_hbm.at[idx_v.at[s].at[pl.ds(j * _L, _L)]],
                    rows_v.at[s].at[pl.ds(j * _L, _L)],
                    sems[s],
                ).wait()

            def tbody(bg, carry):
                row_idx = bg * 16 + lanes
                for dd in range(d):
                    v = plsc.load_gather(
                        rows_v.at[s], [row_idx, jnp.full((16,), dd, jnp.int32)]
                    )
                    trans_v[dd, pl.ds(bg * 16, 16)] = v
                return carry

            lax.fori_loop(0, bw // 16, tbody, 0)
            pltpu.sync_copy(trans_v, out_hbm.at[t, :, pl.ds(b0, bw)])

        start_chunk(0, 0)

        def body(kk, carry):
            t = kk * 2
            for x in range(2):
                start_chunk(t + x + 1, 1 - x)
                finish_chunk(t + x, x)
            return carry

        lax.fori_loop(0, (t_steps - 2) // 2, body, 0)

        start_chunk(t_steps - 1, 1)
        finish_chunk(t_steps - 2, 0)
        finish_chunk(t_steps - 1, 1)

    return k(table, idx_tb)


def kernel(token_ids, embeddings):
    b, t = token_ids.shape
    v, d = embeddings.shape
    idx_tb = token_ids.T.astype(jnp.int32)
    out_tdb = _gather_impl(t, b, d, embeddings, idx_tb)
    return jnp.transpose(out_tdb, (2, 0, 1))


# speedup vs baseline: 1.3044x; 1.3044x over previous
import functools

import jax
import jax.numpy as jnp
from jax import lax
from jax.experimental import pallas as pl
from jax.experimental.pallas import tpu as pltpu
from jax.experimental.pallas import tpu_sc as plsc

_INFO = plsc.get_sparse_core_info()
_NC = _INFO.num_cores
_NS = _INFO.num_subcores
_NW = _NC * _NS
_L = 128


def _gather_impl(t_steps, b, d, table, idx_tb):
    bw = b // _NW
    assert b % _NW == 0 and bw % _L == 0 and t_steps % 2 == 0
    nj = bw // _L

    mesh = plsc.VectorSubcoreMesh(core_axis_name="c", subcore_axis_name="s")

    @functools.partial(
        pl.kernel,
        mesh=mesh,
        compiler_params=pltpu.CompilerParams(
            use_tc_tiling_on_sc=False, needs_layout_passes=False
        ),
        out_type=jax.ShapeDtypeStruct((t_steps, d, b), jnp.float32),
        scratch_types=[
            pltpu.VMEM((2, bw), jnp.int32),
            pltpu.VMEM((2, bw, d), jnp.float32),
            pltpu.VMEM((d, bw), jnp.float32),
            pltpu.SemaphoreType.DMA,
            pltpu.SemaphoreType.DMA,
        ],
    )
    def k(table_hbm, idx_hbm, out_hbm, idx_v, rows_v, trans_v, sem0, sem1):
        sems = (sem0, sem1)
        wid = lax.axis_index("s") * _NC + lax.axis_index("c")
        b0 = wid * bw
        lanes = lax.iota(jnp.int32, 16)

        def start_chunk(t, s):
            pltpu.sync_copy(idx_hbm.at[t, pl.ds(b0, bw)], idx_v.at[s])
            for j in range(nj):
                pltpu.make_async_copy(
                    table_hbm.at[idx_v.at[s].at[pl.ds(j * _L, _L)]],
                    rows_v.at[s].at[pl.ds(j * _L, _L)],
                    sems[s],
                ).start()

        def finish_chunk(t, s):
            for j in range(nj):
                pltpu.make_async_copy(
                    table_hbm.at[idx_v.at[s].at[pl.ds(j * _L, _L)]],
                    rows_v.at[s].at[pl.ds(j * _L, _L)],
                    sems[s],
                ).wait()

            def tbody(bg, carry):
                row_idx = bg * 16 + lanes
                for d0 in range(0, d, 8):
                    vs = [
                        plsc.load_gather(
                            rows_v.at[s],
                            [row_idx, jnp.full((16,), d0 + q, jnp.int32)],
                        )
                        for q in range(8)
                    ]
                    for q in range(8):
                        trans_v[d0 + q, pl.ds(bg * 16, 16)] = vs[q]
                return carry

            lax.fori_loop(0, bw // 16, tbody, 0)
            pltpu.sync_copy(trans_v, out_hbm.at[t, :, pl.ds(b0, bw)])

        start_chunk(0, 0)

        def body(kk, carry):
            t = kk * 2
            for x in range(2):
                start_chunk(t + x + 1, 1 - x)
                finish_chunk(t + x, x)
            return carry

        lax.fori_loop(0, (t_steps - 2) // 2, body, 0)

        start_chunk(t_steps - 1, 1)
        finish_chunk(t_steps - 2, 0)
        finish_chunk(t_steps - 1, 1)

    return k(table, idx_tb)


def kernel(token_ids, embeddings):
    b, t = token_ids.shape
    v, d = embeddings.shape
    idx_tb = token_ids.T.astype(jnp.int32)
    out_tdb = _gather_impl(t, b, d, embeddings, idx_tb)
    return jnp.transpose(out_tdb, (2, 0, 1))


# speedup vs baseline: 1.6874x; 1.2936x over previous
import functools

import jax
import jax.numpy as jnp
from jax import lax
from jax.experimental import pallas as pl
from jax.experimental.pallas import tpu as pltpu
from jax.experimental.pallas import tpu_sc as plsc

_INFO = plsc.get_sparse_core_info()
_NC = _INFO.num_cores
_NS = _INFO.num_subcores
_NW = _NC * _NS
_L = 128


def _gather_impl(t_steps, b, d, table, idx_tb):
    bw = b // _NW
    assert b % _NW == 0 and bw % _L == 0 and t_steps % 2 == 0
    nj = bw // _L

    mesh = plsc.VectorSubcoreMesh(core_axis_name="c", subcore_axis_name="s")

    @functools.partial(
        pl.kernel,
        mesh=mesh,
        compiler_params=pltpu.CompilerParams(use_tc_tiling_on_sc=False),
        out_type=jax.ShapeDtypeStruct((b, t_steps, d), jnp.float32),
        scratch_types=[
            pltpu.VMEM((2, bw), jnp.int32),
            pltpu.VMEM((2, bw, d), jnp.float32),
            pltpu.SemaphoreType.DMA,
            pltpu.SemaphoreType.DMA,
        ],
    )
    def k(table_hbm, idx_hbm, out_hbm, idx_v, rows_v, sem0, sem1):
        sems = (sem0, sem1)
        wid = lax.axis_index("s") * _NC + lax.axis_index("c")
        b0 = wid * bw

        def start_chunk(t, s):
            pltpu.sync_copy(idx_hbm.at[t, pl.ds(b0, bw)], idx_v.at[s])
            for j in range(nj):
                pltpu.make_async_copy(
                    table_hbm.at[idx_v.at[s].at[pl.ds(j * _L, _L)]],
                    rows_v.at[s].at[pl.ds(j * _L, _L)],
                    sems[s],
                ).start()

        def finish_chunk(t, s):
            for j in range(nj):
                pltpu.make_async_copy(
                    table_hbm.at[idx_v.at[s].at[pl.ds(j * _L, _L)]],
                    rows_v.at[s].at[pl.ds(j * _L, _L)],
                    sems[s],
                ).wait()
            pltpu.sync_copy(rows_v.at[s], out_hbm.at[pl.ds(b0, bw), t])

        start_chunk(0, 0)

        def body(kk, carry):
            t = kk * 2
            for x in range(2):
                start_chunk(t + x + 1, 1 - x)
                finish_chunk(t + x, x)
            return carry

        lax.fori_loop(0, (t_steps - 2) // 2, body, 0)

        start_chunk(t_steps - 1, 1)
        finish_chunk(t_steps - 2, 0)
        finish_chunk(t_steps - 1, 1)

    return k(table, idx_tb)


def kernel(token_ids, embeddings):
    b, t = token_ids.shape
    v, d = embeddings.shape
    idx_tb = token_ids.T.astype(jnp.int32)
    return _gather_impl(t, b, d, embeddings, idx_tb)


# speedup vs baseline: 1.7028x; 1.0091x over previous
import functools

import jax
import jax.numpy as jnp
from jax import lax
from jax.experimental import pallas as pl
from jax.experimental.pallas import tpu as pltpu
from jax.experimental.pallas import tpu_sc as plsc

_INFO = plsc.get_sparse_core_info()
_NC = _INFO.num_cores
_NS = _INFO.num_subcores
_NW = _NC * _NS
_L = 128


def _gather_impl(t_steps, b, d, table, idx_tb):
    bw = b // _NW
    assert b % _NW == 0 and bw % _L == 0 and t_steps % 2 == 0
    nj = bw // _L

    mesh = plsc.VectorSubcoreMesh(core_axis_name="c", subcore_axis_name="s")

    @functools.partial(
        pl.kernel,
        mesh=mesh,
        compiler_params=pltpu.CompilerParams(use_tc_tiling_on_sc=False),
        out_type=jax.ShapeDtypeStruct((b, t_steps, d), jnp.float32),
        scratch_types=[
            pltpu.VMEM((t_steps, bw), jnp.int32),
            pltpu.VMEM((2, bw, d), jnp.float32),
            pltpu.SemaphoreType.DMA,
            pltpu.SemaphoreType.DMA,
        ],
    )
    def k(table_hbm, idx_hbm, out_hbm, idx_v, rows_v, sem0, sem1):
        sems = (sem0, sem1)
        wid = lax.axis_index("s") * _NC + lax.axis_index("c")
        b0 = wid * bw
        pltpu.sync_copy(idx_hbm.at[:, pl.ds(b0, bw)], idx_v)

        def start_chunk(t, s):
            for j in range(nj):
                pltpu.make_async_copy(
                    table_hbm.at[idx_v.at[t].at[pl.ds(j * _L, _L)]],
                    rows_v.at[s].at[pl.ds(j * _L, _L)],
                    sems[s],
                ).start()

        def finish_chunk(t, s):
            for j in range(nj):
                pltpu.make_async_copy(
                    table_hbm.at[idx_v.at[t].at[pl.ds(j * _L, _L)]],
                    rows_v.at[s].at[pl.ds(j * _L, _L)],
                    sems[s],
                ).wait()
            pltpu.sync_copy(rows_v.at[s], out_hbm.at[pl.ds(b0, bw), t])

        start_chunk(0, 0)

        def body(kk, carry):
            t = kk * 2
            for x in range(2):
                start_chunk(t + x + 1, 1 - x)
                finish_chunk(t + x, x)
            return carry

        lax.fori_loop(0, (t_steps - 2) // 2, body, 0)

        start_chunk(t_steps - 1, 1)
        finish_chunk(t_steps - 2, 0)
        finish_chunk(t_steps - 1, 1)

    return k(table, idx_tb)


def kernel(token_ids, embeddings):
    b, t = token_ids.shape
    v, d = embeddings.shape
    idx_tb = token_ids.T.astype(jnp.int32)
    return _gather_impl(t, b, d, embeddings, idx_tb)
